# Initial kernel scaffold; baseline (speedup 1.0000x reference)
import jax
import jax.numpy as jnp
from jax.experimental import pallas as pl


def kernel(input_ids, word_emb, pos_emb, gamma, beta):
    raise NotImplementedError("write your pallas kernel here")


# speedup vs baseline: 9.1049x; 9.1049x over previous
import functools

import jax
import jax.numpy as jnp
from jax import lax
from jax.experimental import pallas as pl
from jax.experimental.pallas import tpu as pltpu
from jax.experimental.pallas import tpu_sc as plsc

VOCAB = 100000
HIDDEN = 128
MAX_POS = 512
B, L = 1024, 200
N = B * L
EPS = 1e-12

NUM_WORKERS = 32
ROWS_PER_W = N // NUM_WORKERS
CHUNK = 640
NCHUNK = ROWS_PER_W // CHUNK


def _sc_gather_body(ids_hbm, table_hbm, out_hbm, idx_v, rows_v, sem):
    c = lax.axis_index("c")
    s = lax.axis_index("s")
    wid = s * 2 + c
    base = wid * ROWS_PER_W

    def chunk_step(k, carry):
        off = base + k * CHUNK
        pltpu.sync_copy(ids_hbm.at[pl.ds(off, CHUNK)], idx_v)
        pltpu.async_copy(table_hbm.at[idx_v], rows_v, sem).wait()
        pltpu.sync_copy(rows_v, out_hbm.at[pl.ds(off, CHUNK)])
        return carry

    lax.fori_loop(0, NCHUNK, chunk_step, 0)


@jax.jit
def _sc_gather(ids, table):
    mesh = plsc.VectorSubcoreMesh(core_axis_name="c", subcore_axis_name="s")
    fn = pl.kernel(
        _sc_gather_body,
        out_type=jax.ShapeDtypeStruct((N, HIDDEN), jnp.float32),
        mesh=mesh,
        scratch_types=[
            pltpu.VMEM((CHUNK,), jnp.int32),
            pltpu.VMEM((CHUNK, HIDDEN), jnp.float32),
            pltpu.SemaphoreType.DMA,
        ],
    )
    return fn(ids, table)


def _tc_ln_kernel(x_ref, pos_ref, gamma_ref, beta_ref, out_ref):
    x = x_ref[...] + pos_ref[...][None, :, :]
    mean = jnp.mean(x, axis=-1, keepdims=True)
    xc = x - mean
    var = jnp.mean(xc * xc, axis=-1, keepdims=True)
    y = xc * lax.rsqrt(var + EPS)
    out_ref[...] = y * gamma_ref[...][None, None, :] + beta_ref[...][None, None, :]


@jax.jit
def _tc_ln(x, pos_emb, gamma, beta):
    BB = 64
    grid = (B // BB,)
    return pl.pallas_call(
        _tc_ln_kernel,
        out_shape=jax.ShapeDtypeStruct((B, L, HIDDEN), jnp.float32),
        grid=grid,
        in_specs=[
            pl.BlockSpec((BB, L, HIDDEN), lambda i: (i, 0, 0)),
            pl.BlockSpec((L, HIDDEN), lambda i: (0, 0)),
            pl.BlockSpec((HIDDEN,), lambda i: (0,)),
            pl.BlockSpec((HIDDEN,), lambda i: (0,)),
        ],
        out_specs=pl.BlockSpec((BB, L, HIDDEN), lambda i: (i, 0, 0)),
    )(x, pos_emb, gamma, beta)


def kernel(input_ids, word_emb, pos_emb, gamma, beta):
    ids = input_ids.reshape(-1).astype(jnp.int32)
    gathered = _sc_gather(ids, word_emb)
    x = gathered.reshape(B, L, HIDDEN)
    return _tc_ln(x, pos_emb[:L], gamma, beta)
